# Initial kernel scaffold; baseline (speedup 1.0000x reference)
import jax
import jax.numpy as jnp
from jax.experimental import pallas as pl


def kernel(features, conv_w, conv_b, cls_w, cls_b, bbox_w, bbox_b):
    raise NotImplementedError("write your pallas kernel here")


# speedup vs baseline: 2.3853x; 2.3853x over previous
import jax
import jax.numpy as jnp
import numpy as np
from jax import lax
from jax.experimental import pallas as pl
from jax.experimental.pallas import tpu as pltpu

B, C, HF, WF = 4, 512, 32, 32
A = 12
IMG = 512.0
STRIDE = 16
PRE = 500
NMS_T = 0.7
MIN_SIZE = 1e-3
BBOX_CLIP = float(np.log(1000.0 / 16.0))

WPAD = 34
NROW = 32 * WPAD
NFLAT = 34 * WPAD
NFLAT_PAD = 1184
NANCH = HF * WF * A
KPAD = 512


def _anchors_np():
    scales = np.array([64.0, 128.0, 256.0, 512.0], np.float32)
    ratios = np.array([0.5, 1.0, 2.0], np.float32)
    h_r = np.sqrt(ratios)
    w_r = 1.0 / h_r
    ws = (w_r[:, None] * scales[None, :]).reshape(-1)
    hs = (h_r[:, None] * scales[None, :]).reshape(-1)
    base = np.round(np.stack([-ws / 2.0, -hs / 2.0, ws / 2.0, hs / 2.0], axis=1))
    sx = np.arange(WF, dtype=np.float32) * STRIDE
    sy = np.arange(HF, dtype=np.float32) * STRIDE
    yy, xx = np.meshgrid(sy, sx, indexing="ij")
    shifts = np.stack([xx.reshape(-1), yy.reshape(-1), xx.reshape(-1), yy.reshape(-1)], axis=1)
    return (shifts[:, None, :] + base[None, :, :]).reshape(-1, 4).astype(np.float32)


def _conv_head_body(x_ref, wt_ref, cb_ref, wh_ref, hb_ref, out_ref):
    acc = jnp.zeros((NROW, C), jnp.float32)
    for t in range(9):
        off = WPAD * (t // 3) + (t % 3)
        acc = acc + jnp.dot(x_ref[0, pl.ds(off, NROW), :].astype(jnp.bfloat16),
                            wt_ref[t].astype(jnp.bfloat16),
                            preferred_element_type=jnp.float32)
    trelu = jnp.maximum(acc + cb_ref[...], 0.0)
    heads = jnp.dot(trelu.astype(jnp.bfloat16), wh_ref[...].astype(jnp.bfloat16),
                    preferred_element_type=jnp.float32) + hb_ref[...]
    out_ref[0] = heads


def _iota2(shape, dim):
    return lax.broadcasted_iota(jnp.int32, shape, dim).astype(jnp.float32)


def _select_nms_body(obj_ref, m_ref, out_ref, slot_ref, acc_ref):
    obj = obj_ref[0]
    si = lax.bitcast_convert_type(obj, jnp.int32)
    key = jnp.bitwise_xor(
        si, jnp.bitwise_and(jnp.right_shift(si, 31), jnp.int32(0x7FFFFFFF)))

    def count_ge(t):
        return jnp.sum((key >= t).astype(jnp.int32))

    base0 = jnp.where(count_ge(jnp.int32(0)) >= PRE, jnp.int32(0),
                      jnp.int32(-2147483648))

    def bit_body(t, base):
        cand = jnp.bitwise_or(base, lax.shift_left(jnp.int32(1), 30 - t))
        return jnp.where(count_ge(cand) >= PRE, cand, base)

    thr = lax.fori_loop(0, 31, bit_body, base0)

    gt = key > thr
    eq = key == thr
    gt_f = gt.astype(jnp.float32)
    eq_f = eq.astype(jnp.float32)
    ltinc = (_iota2((128, 128), 0) <= _iota2((128, 128), 1)).astype(jnp.float32)
    stl96 = (_iota2((96, 96), 1) < _iota2((96, 96), 0)).astype(jnp.float32)

    def excl_prefix(f):
        rowcum = jnp.dot(f, ltinc, preferred_element_type=jnp.float32, precision=lax.Precision.HIGHEST)
        rowtot = jnp.sum(f, axis=1, keepdims=True)
        off = jnp.dot(stl96, rowtot, preferred_element_type=jnp.float32, precision=lax.Precision.HIGHEST)
        return rowcum + off - f

    excl_gt = excl_prefix(gt_f)
    excl_eq = excl_prefix(eq_f)
    cnt_gt = jnp.sum(gt_f)
    slot_eq = cnt_gt + excl_eq
    sel = jnp.logical_or(gt, jnp.logical_and(eq, slot_eq < float(PRE)))
    slot = jnp.where(gt, excl_gt, slot_eq)
    slot_ref[...] = jnp.where(sel, slot, -1.0)

    acc_ref[...] = jnp.zeros((KPAD, 16), jnp.float32)
    kiota = _iota2((KPAD, 128), 0)

    def chunk_body(r, carry):
        rows = slot_ref[pl.ds(r * 8, 8), :]
        for q in range(8):
            p = (rows[q:q + 1, :] == kiota).astype(jnp.float32)
            mrows = m_ref[0, pl.ds((r * 8 + q) * 128, 128), :]
            acc_ref[...] += jnp.dot(p, mrows, preferred_element_type=jnp.float32, precision=lax.Precision.HIGHEST)
        return carry

    lax.fori_loop(0, 12, chunk_body, 0)
    acc = acc_ref[...]

    kcol = _iota2((KPAD, 1), 0)
    pad_rows = kcol >= float(PRE)
    l_rank = jnp.where(pad_rows, -1e30, acc[:, 0:1])
    i_rank = jnp.where(pad_rows, 1e6 + kcol, acc[:, 1:2])

    eye = (_iota2((KPAD, KPAD), 0) == _iota2((KPAD, KPAD), 1)).astype(jnp.float32)

    def trsp(x):
        return lax.dot_general(x, eye, (((0,), (0,)), ((), ())),
                               preferred_element_type=jnp.float32, precision=lax.Precision.HIGHEST)

    li = jnp.concatenate([l_rank, i_rank], axis=1)
    lir = trsp(li)
    lr, ir = lir[0:1, :], lir[1:2, :]
    bigger = jnp.logical_or(lr > l_rank,
                            jnp.logical_and(lr == l_rank, ir < i_rank))
    rank = jnp.sum(bigger.astype(jnp.float32), axis=1, keepdims=True)
    rankr = trsp(rank)
    qmat = (rankr == _iota2((KPAD, KPAD), 0)).astype(jnp.float32)
    srt = jnp.dot(qmat, acc, preferred_element_type=jnp.float32, precision=lax.Precision.HIGHEST)

    ls = srt[:, 0:1]
    d0, d1, d2, d3 = srt[:, 2:3], srt[:, 3:4], srt[:, 4:5], srt[:, 5:6]
    a0, a1, a2, a3 = srt[:, 6:7], srt[:, 7:8], srt[:, 8:9], srt[:, 9:10]

    score = 1.0 / (1.0 + jnp.exp(-ls))
    wa = a2 - a0
    ha = a3 - a1
    cx = a0 + 0.5 * wa
    cy = a1 + 0.5 * ha
    dw = jnp.minimum(d2, BBOX_CLIP)
    dh = jnp.minimum(d3, BBOX_CLIP)
    pcx = d0 * wa + cx
    pcy = d1 * ha + cy
    pw = jnp.exp(dw) * wa
    ph = jnp.exp(dh) * ha
    x1 = jnp.clip(pcx - 0.5 * pw, 0.0, IMG)
    y1 = jnp.clip(pcy - 0.5 * ph, 0.0, IMG)
    x2 = jnp.clip(pcx + 0.5 * pw, 0.0, IMG)
    y2 = jnp.clip(pcy + 0.5 * ph, 0.0, IMG)
    valid = jnp.logical_and(
        jnp.logical_and(x2 - x1 >= MIN_SIZE, y2 - y1 >= MIN_SIZE),
        jnp.logical_not(pad_rows))
    v_f = valid.astype(jnp.float32)

    stl = (_iota2((KPAD, KPAD), 1) < _iota2((KPAD, KPAD), 0)).astype(jnp.float32)
    exclv = jnp.dot(stl, v_f, preferred_element_type=jnp.float32, precision=lax.Precision.HIGHEST)
    cntv = jnp.sum(v_f)
    pos = jnp.where(valid, exclv, cntv + kcol - exclv)
    posr = trsp(pos)
    rmat = (posr == _iota2((KPAD, KPAD), 0)).astype(jnp.float32)
    fields = jnp.concatenate([x1, y1, x2, y2, score, v_f], axis=1)
    part = jnp.dot(rmat, fields, preferred_element_type=jnp.float32, precision=lax.Precision.HIGHEST)

    px1, py1 = part[:, 0:1], part[:, 1:2]
    px2, py2 = part[:, 2:3], part[:, 3:4]
    ps, pv = part[:, 4:5], part[:, 5:6]
    area = (px2 - px1) * (py2 - py1)
    rowsr = trsp(jnp.concatenate([px1, py1, px2, py2, area], axis=1))
    x1r, y1r = rowsr[0:1, :], rowsr[1:2, :]
    x2r, y2r, arear = rowsr[2:3, :], rowsr[3:4, :], rowsr[4:5, :]
    xx1 = jnp.maximum(px1, x1r)
    yy1 = jnp.maximum(py1, y1r)
    xx2 = jnp.minimum(px2, x2r)
    yy2 = jnp.minimum(py2, y2r)
    inter = jnp.maximum(xx2 - xx1, 0.0) * jnp.maximum(yy2 - yy1, 0.0)
    iou = inter / (area + arear - inter + 1e-9)
    smat = jnp.logical_and(iou > NMS_T,
                           _iota2((KPAD, KPAD), 1) < _iota2((KPAD, KPAD), 0)
                           ).astype(jnp.float32)

    def nms_cond(carry):
        _, changed, it = carry
        return jnp.logical_and(changed, it < KPAD + 2)

    def nms_body(carry):
        keep, _, it = carry
        sup = jnp.dot(smat, keep, preferred_element_type=jnp.float32, precision=lax.Precision.HIGHEST)
        keepn = jnp.where(sup > 0.5, 0.0, pv)
        changed = jnp.sum(jnp.abs(keepn - keep)) > 0.0
        return keepn, changed, it + 1

    keep, _, _ = lax.while_loop(nms_cond, nms_body,
                                (pv, jnp.bool_(True), jnp.int32(0)))

    fpos = jnp.dot(stl, keep, preferred_element_type=jnp.float32, precision=lax.Precision.HIGHEST)
    fposr = trsp(fpos)
    keepr = trsp(keep)
    fmat = (fposr == _iota2((KPAD, KPAD), 0)).astype(jnp.float32) * keepr
    outf = jnp.concatenate(
        [px1, py1, px2, py2, ps, jnp.zeros((KPAD, 3), jnp.float32)], axis=1)
    out_ref[0] = jnp.dot(fmat, outf, preferred_element_type=jnp.float32, precision=lax.Precision.HIGHEST)


def kernel(features, conv_w, conv_b, cls_w, cls_b, bbox_w, bbox_b):
    x = features.transpose(0, 2, 3, 1)
    x = jnp.pad(x, ((0, 0), (1, 1), (1, 1), (0, 0)))
    x = x.reshape(B, NFLAT, C)
    x = jnp.pad(x, ((0, 0), (0, NFLAT_PAD - NFLAT), (0, 0)))
    wt = conv_w.transpose(2, 3, 1, 0).reshape(9, C, C)
    whead = jnp.concatenate([cls_w.reshape(A, C).T,
                             bbox_w.reshape(4 * A, C).T], axis=1)
    hbias = jnp.concatenate([cls_b, bbox_b]).reshape(1, 60)
    cbias = conv_b.reshape(1, C)

    heads = pl.pallas_call(
        _conv_head_body,
        grid=(B,),
        in_specs=[
            pl.BlockSpec((1, NFLAT_PAD, C), lambda b: (b, 0, 0)),
            pl.BlockSpec((9, C, C), lambda b: (0, 0, 0)),
            pl.BlockSpec((1, C), lambda b: (0, 0)),
            pl.BlockSpec((C, 60), lambda b: (0, 0)),
            pl.BlockSpec((1, 60), lambda b: (0, 0)),
        ],
        out_specs=pl.BlockSpec((1, NROW, 60), lambda b: (b, 0, 0)),
        out_shape=jax.ShapeDtypeStruct((B, NROW, 60), jnp.float32),
    )(x, wt, cbias, whead, hbias)

    hv = heads.reshape(B, HF, WPAD, 60)[:, :, :WF, :]
    obj = hv[..., :A].reshape(B, NANCH)
    deltas = hv[..., A:].reshape(B, NANCH, 4)
    anchors = jnp.asarray(_anchors_np())
    idxcol = jnp.arange(NANCH, dtype=jnp.float32).reshape(1, NANCH, 1)
    m = jnp.concatenate([
        obj[:, :, None],
        jnp.broadcast_to(idxcol, (B, NANCH, 1)),
        deltas,
        jnp.broadcast_to(anchors[None], (B, NANCH, 4)),
        jnp.zeros((B, NANCH, 6), jnp.float32),
    ], axis=2)
    obj2 = obj.reshape(B, 96, 128)

    res = pl.pallas_call(
        _select_nms_body,
        grid=(B,),
        in_specs=[
            pl.BlockSpec((1, 96, 128), lambda b: (b, 0, 0)),
            pl.BlockSpec((1, NANCH, 16), lambda b: (b, 0, 0)),
        ],
        out_specs=pl.BlockSpec((1, KPAD, 8), lambda b: (b, 0, 0)),
        out_shape=jax.ShapeDtypeStruct((B, KPAD, 8), jnp.float32),
        scratch_shapes=[
            pltpu.VMEM((96, 128), jnp.float32),
            pltpu.VMEM((KPAD, 16), jnp.float32),
        ],
    )(obj2, m)

    proposals = res[:, :PRE, :4]
    final_scores = res[:, :PRE, 4]
    return proposals, final_scores
